# Initial kernel scaffold; baseline (speedup 1.0000x reference)
import jax
import jax.numpy as jnp
from jax.experimental import pallas as pl


def kernel(logits, target):
    raise NotImplementedError("write your pallas kernel here")


# speedup vs baseline: 2.0150x; 2.0150x over previous
import functools

import jax
import jax.numpy as jnp
from jax.experimental import pallas as pl
from jax.experimental.pallas import tpu as pltpu

_SMOOTHING = 0.1
_IGNORE_INDEX = -100

_ROW_BLOCK = 512
_CHUNK = 2048


def _loss_body(C, N, t_ref, x_ref, out_ref, m_ref, s_ref, tot_ref, tgt_ref):
    r = pl.program_id(0)
    v = pl.program_id(1)
    nv = pl.num_programs(1)

    @pl.when(v == 0)
    def _init():
        m_ref[...] = jnp.full_like(m_ref, -jnp.inf)
        s_ref[...] = jnp.zeros_like(s_ref)
        tot_ref[...] = jnp.zeros_like(tot_ref)
        tgt_ref[...] = jnp.zeros_like(tgt_ref)

    x = x_ref[...]
    t = t_ref[...]
    cols = v * _CHUNK + jax.lax.broadcasted_iota(jnp.int32, x.shape, 1)
    valid = cols < C
    x_valid = jnp.where(valid, x, -jnp.inf)

    cm = jnp.max(x_valid, axis=1, keepdims=True)
    m_old = m_ref[...]
    m_new = jnp.maximum(m_old, cm)
    alpha = jnp.exp(m_old - m_new)
    e = jnp.exp(x_valid - m_new)
    s_ref[...] = s_ref[...] * alpha + jnp.sum(e, axis=1, keepdims=True)
    tot_ref[...] += jnp.sum(jnp.where(valid, x, 0.0), axis=1, keepdims=True)
    tgt_ref[...] += jnp.sum(jnp.where(cols == t, x, 0.0), axis=1, keepdims=True)
    m_ref[...] = m_new

    @pl.when(v == nv - 1)
    def _finalize():
        lse = m_ref[...] + jnp.log(s_ref[...])
        lp_t = tgt_ref[...] - lse
        sum_lp = tot_ref[...] - jnp.float32(C) * lse
        loss = -((1.0 - _SMOOTHING) * lp_t
                 + (_SMOOTHING / (C - 1)) * (sum_lp - lp_t))
        loss = jnp.where(t == _IGNORE_INDEX, 0.0, loss)
        part = jnp.sum(loss) * (1.0 / N)

        @pl.when(r == 0)
        def _first():
            out_ref[0, 0] = part

        @pl.when(r > 0)
        def _rest():
            out_ref[0, 0] += part


def kernel(logits, target):
    N, C = logits.shape
    nr = N // _ROW_BLOCK
    nv = pl.cdiv(C, _CHUNK)
    t2d = target.reshape(N, 1)

    out = pl.pallas_call(
        functools.partial(_loss_body, C, N),
        grid=(nr, nv),
        in_specs=[
            pl.BlockSpec((_ROW_BLOCK, 1), lambda r, v: (r, 0)),
            pl.BlockSpec((_ROW_BLOCK, _CHUNK), lambda r, v: (r, v)),
        ],
        out_specs=pl.BlockSpec(
            (1, 1), lambda r, v: (0, 0), memory_space=pltpu.SMEM),
        out_shape=jax.ShapeDtypeStruct((1, 1), jnp.float32),
        scratch_shapes=[
            pltpu.VMEM((_ROW_BLOCK, 1), jnp.float32),
            pltpu.VMEM((_ROW_BLOCK, 1), jnp.float32),
            pltpu.VMEM((_ROW_BLOCK, 1), jnp.float32),
            pltpu.VMEM((_ROW_BLOCK, 1), jnp.float32),
        ],
    )(t2d, logits)
    return out[0, 0]


# speedup vs baseline: 2.0840x; 1.0342x over previous
import functools

import jax
import jax.numpy as jnp
from jax.experimental import pallas as pl
from jax.experimental.pallas import tpu as pltpu

_SMOOTHING = 0.1
_IGNORE_INDEX = -100

_ROW_BLOCK = 32


def _loss_body(C, N, t_ref, x_ref, out_ref):
    r = pl.program_id(0)
    x = x_ref[...]
    t = t_ref[...]

    m = jnp.max(x, axis=1, keepdims=True)
    e = jnp.exp(x - m)
    s = jnp.sum(e, axis=1, keepdims=True)
    tot = jnp.sum(x, axis=1, keepdims=True)
    cols = jax.lax.broadcasted_iota(jnp.int32, x.shape, 1)
    tgt = jnp.sum(jnp.where(cols == t, x, 0.0), axis=1, keepdims=True)

    lse = m + jnp.log(s)
    lp_t = tgt - lse
    sum_lp = tot - jnp.float32(C) * lse
    loss = -((1.0 - _SMOOTHING) * lp_t
             + (_SMOOTHING / (C - 1)) * (sum_lp - lp_t))
    loss = jnp.where(t == _IGNORE_INDEX, 0.0, loss)
    part = jnp.sum(loss) * (1.0 / N)

    @pl.when(r == 0)
    def _first():
        out_ref[0, 0] = part

    @pl.when(r > 0)
    def _rest():
        out_ref[0, 0] += part


def kernel(logits, target):
    N, C = logits.shape
    nr = N // _ROW_BLOCK
    t2d = target.reshape(N, 1)

    out = pl.pallas_call(
        functools.partial(_loss_body, C, N),
        grid=(nr,),
        in_specs=[
            pl.BlockSpec((_ROW_BLOCK, 1), lambda r: (r, 0)),
            pl.BlockSpec((_ROW_BLOCK, C), lambda r: (r, 0)),
        ],
        out_specs=pl.BlockSpec(
            (1, 1), lambda r: (0, 0), memory_space=pltpu.SMEM),
        out_shape=jax.ShapeDtypeStruct((1, 1), jnp.float32),
    )(t2d, logits)
    return out[0, 0]
